# baseline (speedup 1.0000x reference)
import functools
import math

import numpy as np
import jax
import jax.numpy as jnp
from jax import lax
from jax.experimental import pallas as pl
from jax.experimental.pallas import tpu as pltpu
from jax.experimental.pallas import tpu_sc as plsc

_BATCH, _SEQ = 16, 8192
_N = _BATCH * _SEQ
_TEMPERATURE = 10000.0
_SCALE = 2.0 * math.pi
_R_MAX = 6000.0
_NUM_R_BINS = 50
_NUM_PHI_BINS = 36
_NUM_FUSED = _NUM_R_BINS * _NUM_PHI_BINS

_NC, _NS = 2, 16
_NW = _NC * _NS
_BPW = _N // _NW
_CH = 128
_NCH = _BPW // _CH
_NBUF = 4
_GAH = 2
_D0 = _NBUF - _GAH

_RA = 64
_BN_SINE = 1024

_ATAN_C = (0.9999798536300659, -0.3326554298400879, 0.1936698853969574,
           -0.11664997786283493, 0.05282219499349594, -0.011769973672926426)
_SIN_C = (6.278553009033203, -41.0910758972168, 77.90902709960938,
          -56.037471771240234)
_RND = 12582912.0


def _sine_consts():
    i = np.arange(64)
    dim_t = _TEMPERATURE ** (2.0 * np.floor(i / 2.0) / 64.0)
    w_half = 1.0 / dim_t
    ph_half = np.where(i % 2 == 1, 0.25, 0.0)
    w = np.concatenate([w_half, w_half])
    ph = np.concatenate([ph_half, ph_half])
    ax = np.where(np.arange(128) < 64, w / 6000.0, 0.0)
    ay = np.where(np.arange(128) < 64, 0.0, w / 4000.0)
    cc = w / 2.0
    return np.stack([ax, ay, cc, w, ph]).astype(np.float32)


_WP_CONST = _sine_consts()


def _bins_body(p_ref, idx_ref):
    p = p_ref[...].reshape(_RA, 128, 2)
    x = p[:, :, 0]
    y = p[:, :, 1]
    s = x * x + y * y
    i = lax.bitcast_convert_type(s, jnp.int32)
    i = 0x5F3759DF - lax.shift_right_logical(i, 1)
    g = lax.bitcast_convert_type(i, jnp.float32)
    hs = 0.5 * s
    g = g * (1.5 - hs * g * g)
    g = g * (1.5 - hs * g * g)
    r = s * g
    rb = jnp.clip((r * (49.0 / _R_MAX)).astype(jnp.int32), 0, 49)

    ax = jnp.abs(x)
    ay = jnp.abs(y)
    hi = jnp.maximum(ax, ay)
    lo = jnp.minimum(ax, ay)
    rc = pl.reciprocal(hi, approx=True)
    rc = rc * (2.0 - hi * rc)
    t = lo * rc
    z = t * t
    a = _ATAN_C[5]
    for k in (4, 3, 2, 1, 0):
        a = a * z + _ATAN_C[k]
    a = a * t
    a = jnp.where(ay > ax, (math.pi / 2) - a, a)
    a = jnp.where(x < 0.0, math.pi - a, a)
    phi = jnp.where(y < 0.0, -a, a)
    pb = ((phi + math.pi) * (35.0 / (2.0 * math.pi))).astype(jnp.int32)
    pb = jnp.clip(pb, 0, 35)
    idx_ref[...] = rb * _NUM_PHI_BINS + pb


def _sine_body(_, pos_ref, wp_ref, out_ref):
    p = pos_ref[...]
    a2 = wp_ref[0:2, :]
    cc = wp_ref[2]
    wb = wp_ref[3]
    ph = wp_ref[4]
    th = lax.dot_general(p, a2, (((1,), (0,)), ((), ())),
                         preferred_element_type=jnp.float32)
    th = jnp.minimum(jnp.maximum(th + cc[None, :], 0.0), wb[None, :]) + ph[None, :]
    u = th - ((th + _RND) - _RND)
    z = u * u
    sv = _SIN_C[3]
    for k in (2, 1, 0):
        sv = sv * z + _SIN_C[k]
    out_ref[...] = sv * u


@functools.cache
def _make_sc_gather():
    mesh = plsc.VectorSubcoreMesh(core_axis_name="c", subcore_axis_name="s")
    return functools.partial(
        pl.kernel,
        out_type=jax.ShapeDtypeStruct((_N, 256), jnp.float32),
        mesh=mesh,
        scratch_types=[
            pltpu.VMEM_SHARED((_NUM_FUSED, 128), jnp.float32),
            pltpu.VMEM((_NCH, _CH), jnp.int32),
            pltpu.VMEM((_NBUF, _CH, 128), jnp.float32),
            pltpu.SemaphoreType.DMA,
            pltpu.SemaphoreType.DMA,
        ],
    )(_sc_gather_body)


def _sc_gather_body(table_hbm, idx_hbm, out_hbm, tab_sh, idx_v, rows_v, gsem, ssem):
    sid = lax.axis_index("s")
    wid = sid * _NC + lax.axis_index("c")
    row0 = wid * _BPW

    @pl.when(sid == 0)
    def _():
        pltpu.sync_copy(table_hbm, tab_sh)

    pltpu.sync_copy(idx_hbm.at[pl.ds(wid * _NCH, _NCH)], idx_v)
    plsc.subcore_barrier()

    def _gather(c, b):
        pltpu.async_copy(tab_sh.at[idx_v.at[c]], rows_v.at[b], gsem)

    def _gather_wait(c, b):
        pltpu.make_async_copy(tab_sh.at[idx_v.at[c]], rows_v.at[b], gsem).wait()

    def _store(c, b):
        dst = out_hbm.at[pl.ds(row0 + c * _CH, _CH), pl.ds(128, 128)]
        pltpu.async_copy(rows_v.at[b], dst, ssem)

    def _store_drain():
        dst = out_hbm.at[pl.ds(row0, _CH), pl.ds(128, 128)]
        pltpu.make_async_copy(rows_v.at[0], dst, ssem).wait()

    for j in range(_GAH):
        _gather(j, j)

    @pl.loop(0, _NCH, step=_NBUF)
    def _chunks(c0):
        for b in range(_NBUF):
            cc = c0 + b
            g = cc + _GAH

            @pl.when(cc >= _D0)
            def _():
                _store_drain()

            @pl.when(g < _NCH)
            def _():
                _gather(g, (b + _GAH) % _NBUF)

            _gather_wait(cc, b)
            _store(cc, b)

    for _ in range(_D0):
        _store_drain()


def _fused_table(r_embed, phi_embed):
    return jnp.concatenate(
        [
            jnp.broadcast_to(r_embed[:, None, :], (_NUM_R_BINS, _NUM_PHI_BINS, 64)),
            jnp.broadcast_to(phi_embed[None, :, :], (_NUM_R_BINS, _NUM_PHI_BINS, 64)),
        ],
        axis=-1,
    ).reshape(_NUM_FUSED, 128)


def kernel(positions, r_embed, phi_embed):
    pos2 = positions.reshape(_N, 2)
    posi = positions.reshape(_N // 128, 256)

    idx2 = pl.pallas_call(
        _bins_body,
        grid=(_N // (_RA * 128),),
        in_specs=[pl.BlockSpec((_RA, 256), lambda i: (i, 0))],
        out_specs=pl.BlockSpec((_RA, 128), lambda i: (i, 0)),
        out_shape=jax.ShapeDtypeStruct((_N // 128, 128), jnp.int32),
    )(posi)

    comb = _make_sc_gather()(_fused_table(r_embed, phi_embed), idx2)

    comb = pl.pallas_call(
        _sine_body,
        grid=(_N // _BN_SINE,),
        in_specs=[
            pl.BlockSpec(memory_space=pl.ANY),
            pl.BlockSpec((_BN_SINE, 2), lambda i: (i, 0)),
            pl.BlockSpec((5, 128), lambda i: (0, 0)),
        ],
        out_specs=pl.BlockSpec((_BN_SINE, 128), lambda i: (i, 0)),
        out_shape=jax.ShapeDtypeStruct((_N, 256), jnp.float32),
        input_output_aliases={0: 0},
    )(comb, pos2, jnp.asarray(_WP_CONST))

    return comb.reshape(_BATCH, _SEQ, 256)


# speedup vs baseline: 2.4332x; 2.4332x over previous
import functools
import math

import numpy as np
import jax
import jax.numpy as jnp
from jax import lax
from jax.experimental import pallas as pl
from jax.experimental.pallas import tpu as pltpu
from jax.experimental.pallas import tpu_sc as plsc

_BATCH, _SEQ = 16, 8192
_N = _BATCH * _SEQ
_TEMPERATURE = 10000.0
_SCALE = 2.0 * math.pi
_R_MAX = 6000.0
_NUM_R_BINS = 50
_NUM_PHI_BINS = 36
_NUM_FUSED = _NUM_R_BINS * _NUM_PHI_BINS

_NC, _NS = 2, 16
_NW = _NC * _NS
_BPW = _N // _NW
_CH = 128
_NCH = _BPW // _CH
_NBUF = 4
_GAH = 2
_D0 = _NBUF - _GAH

_RA = 64
_BN_SINE = 1024

_ATAN_C = (0.9999798536300659, -0.3326554298400879, 0.1936698853969574,
           -0.11664997786283493, 0.05282219499349594, -0.011769973672926426)
_SIN_C = (6.278553009033203, -41.0910758972168, 77.90902709960938,
          -56.037471771240234)
_RND = 12582912.0


def _sine_consts():
    i = np.arange(64)
    dim_t = _TEMPERATURE ** (2.0 * np.floor(i / 2.0) / 64.0)
    w_half = 1.0 / dim_t
    ph_half = np.where(i % 2 == 1, 0.25, 0.0)
    w = np.concatenate([w_half, w_half])
    ph = np.concatenate([ph_half, ph_half])
    ax = np.where(np.arange(128) < 64, w / 6000.0, 0.0)
    ay = np.where(np.arange(128) < 64, 0.0, w / 4000.0)
    cc = w / 2.0
    return np.stack([ax, ay, cc, w, ph]).astype(np.float32)


_WP_CONST = _sine_consts()


def _bins_body(x_ref, y_ref, idx_ref):
    x = x_ref[...]
    y = y_ref[...]
    s = x * x + y * y
    i = lax.bitcast_convert_type(s, jnp.int32)
    i = 0x5F3759DF - lax.shift_right_logical(i, 1)
    g = lax.bitcast_convert_type(i, jnp.float32)
    hs = 0.5 * s
    g = g * (1.5 - hs * g * g)
    g = g * (1.5 - hs * g * g)
    r = s * g
    rb = jnp.clip((r * (49.0 / _R_MAX)).astype(jnp.int32), 0, 49)

    ax = jnp.abs(x)
    ay = jnp.abs(y)
    hi = jnp.maximum(ax, ay)
    lo = jnp.minimum(ax, ay)
    rc = pl.reciprocal(hi, approx=True)
    rc = rc * (2.0 - hi * rc)
    t = lo * rc
    z = t * t
    a = _ATAN_C[5]
    for k in (4, 3, 2, 1, 0):
        a = a * z + _ATAN_C[k]
    a = a * t
    a = jnp.where(ay > ax, (math.pi / 2) - a, a)
    a = jnp.where(x < 0.0, math.pi - a, a)
    phi = jnp.where(y < 0.0, -a, a)
    pb = ((phi + math.pi) * (35.0 / (2.0 * math.pi))).astype(jnp.int32)
    pb = jnp.clip(pb, 0, 35)
    idx_ref[...] = rb * _NUM_PHI_BINS + pb


def _sine_body(_, pos_ref, wp_ref, out_ref):
    p = pos_ref[...]
    a2 = wp_ref[0:2, :]
    cc = wp_ref[2]
    wb = wp_ref[3]
    ph = wp_ref[4]
    th = lax.dot_general(p, a2, (((1,), (0,)), ((), ())),
                         preferred_element_type=jnp.float32)
    th = jnp.minimum(jnp.maximum(th + cc[None, :], 0.0), wb[None, :]) + ph[None, :]
    u = th - ((th + _RND) - _RND)
    z = u * u
    sv = _SIN_C[3]
    for k in (2, 1, 0):
        sv = sv * z + _SIN_C[k]
    out_ref[...] = sv * u


@functools.cache
def _make_sc_gather():
    mesh = plsc.VectorSubcoreMesh(core_axis_name="c", subcore_axis_name="s")
    return functools.partial(
        pl.kernel,
        out_type=jax.ShapeDtypeStruct((_N, 256), jnp.float32),
        mesh=mesh,
        scratch_types=[
            pltpu.VMEM_SHARED((_NUM_FUSED, 128), jnp.float32),
            pltpu.VMEM((_NCH, _CH), jnp.int32),
            pltpu.VMEM((_NBUF, _CH, 128), jnp.float32),
            pltpu.SemaphoreType.DMA,
            pltpu.SemaphoreType.DMA,
        ],
    )(_sc_gather_body)


def _sc_gather_body(table_hbm, idx_hbm, out_hbm, tab_sh, idx_v, rows_v, gsem, ssem):
    sid = lax.axis_index("s")
    wid = sid * _NC + lax.axis_index("c")
    row0 = wid * _BPW

    @pl.when(sid == 0)
    def _():
        pltpu.sync_copy(table_hbm, tab_sh)

    pltpu.sync_copy(idx_hbm.at[pl.ds(wid * _NCH, _NCH)], idx_v)
    plsc.subcore_barrier()

    def _gather(c, b):
        pltpu.async_copy(tab_sh.at[idx_v.at[c]], rows_v.at[b], gsem)

    def _gather_wait(c, b):
        pltpu.make_async_copy(tab_sh.at[idx_v.at[c]], rows_v.at[b], gsem).wait()

    def _store(c, b):
        dst = out_hbm.at[pl.ds(row0 + c * _CH, _CH), pl.ds(128, 128)]
        pltpu.async_copy(rows_v.at[b], dst, ssem)

    def _store_drain():
        dst = out_hbm.at[pl.ds(row0, _CH), pl.ds(128, 128)]
        pltpu.make_async_copy(rows_v.at[0], dst, ssem).wait()

    for j in range(_GAH):
        _gather(j, j)

    @pl.loop(0, _NCH, step=_NBUF)
    def _chunks(c0):
        for b in range(_NBUF):
            cc = c0 + b
            g = cc + _GAH

            @pl.when(cc >= _D0)
            def _():
                _store_drain()

            @pl.when(g < _NCH)
            def _():
                _gather(g, (b + _GAH) % _NBUF)

            _gather_wait(cc, b)
            _store(cc, b)

    for _ in range(_D0):
        _store_drain()


def _fused_table(r_embed, phi_embed):
    return jnp.concatenate(
        [
            jnp.broadcast_to(r_embed[:, None, :], (_NUM_R_BINS, _NUM_PHI_BINS, 64)),
            jnp.broadcast_to(phi_embed[None, :, :], (_NUM_R_BINS, _NUM_PHI_BINS, 64)),
        ],
        axis=-1,
    ).reshape(_NUM_FUSED, 128)


def kernel(positions, r_embed, phi_embed):
    pos2 = positions.reshape(_N, 2)
    xcol = positions[..., 0].reshape(_N // 128, 128)
    ycol = positions[..., 1].reshape(_N // 128, 128)

    idx2 = pl.pallas_call(
        _bins_body,
        grid=(_N // (_RA * 128),),
        in_specs=[
            pl.BlockSpec((_RA, 128), lambda i: (i, 0)),
            pl.BlockSpec((_RA, 128), lambda i: (i, 0)),
        ],
        out_specs=pl.BlockSpec((_RA, 128), lambda i: (i, 0)),
        out_shape=jax.ShapeDtypeStruct((_N // 128, 128), jnp.int32),
    )(xcol, ycol)

    comb = _make_sc_gather()(_fused_table(r_embed, phi_embed), idx2)

    comb = pl.pallas_call(
        _sine_body,
        grid=(_N // _BN_SINE,),
        in_specs=[
            pl.BlockSpec(memory_space=pl.ANY),
            pl.BlockSpec((_BN_SINE, 2), lambda i: (i, 0)),
            pl.BlockSpec((5, 128), lambda i: (0, 0)),
        ],
        out_specs=pl.BlockSpec((_BN_SINE, 128), lambda i: (i, 0)),
        out_shape=jax.ShapeDtypeStruct((_N, 256), jnp.float32),
        input_output_aliases={0: 0},
    )(comb, pos2, jnp.asarray(_WP_CONST))

    return comb.reshape(_BATCH, _SEQ, 256)


# speedup vs baseline: 2.9266x; 1.2028x over previous
import functools
import math

import numpy as np
import jax
import jax.numpy as jnp
from jax import lax
from jax.experimental import pallas as pl
from jax.experimental.pallas import tpu as pltpu
from jax.experimental.pallas import tpu_sc as plsc

_BATCH, _SEQ = 16, 8192
_N = _BATCH * _SEQ
_TEMPERATURE = 10000.0
_SCALE = 2.0 * math.pi
_R_MAX = 6000.0
_NUM_R_BINS = 50
_NUM_PHI_BINS = 36
_NUM_FUSED = _NUM_R_BINS * _NUM_PHI_BINS

_NC, _NS = 2, 16
_NW = _NC * _NS
_BPW = _N // _NW
_CH = 128
_NCH = _BPW // _CH
_NBUF = 4
_GAH = 2
_D0 = _NBUF - _GAH

_RA = 64
_BN_SINE = 2048

_ATAN_C = (0.9999798536300659, -0.3326554298400879, 0.1936698853969574,
           -0.11664997786283493, 0.05282219499349594, -0.011769973672926426)
_SIN_C = (6.278553009033203, -41.0910758972168, 77.90902709960938,
          -56.037471771240234)
_RND = 12582912.0


def _sine_consts():
    i = np.arange(64)
    dim_t = _TEMPERATURE ** (2.0 * np.floor(i / 2.0) / 64.0)
    w_half = 1.0 / dim_t
    ph_half = np.where(i % 2 == 1, 0.25, 0.0)
    w = np.concatenate([w_half, w_half])
    ph = np.concatenate([ph_half, ph_half])
    ax = np.where(np.arange(128) < 64, w / 6000.0, 0.0)
    ay = np.where(np.arange(128) < 64, 0.0, w / 4000.0)
    cc = w / 2.0
    return np.stack([ax, ay, cc, w, ph]).astype(np.float32)


_WP_CONST = _sine_consts()


def _bins_body(x_ref, y_ref, idx_ref):
    x = x_ref[...]
    y = y_ref[...]
    s = x * x + y * y
    i = lax.bitcast_convert_type(s, jnp.int32)
    i = 0x5F3759DF - lax.shift_right_logical(i, 1)
    g = lax.bitcast_convert_type(i, jnp.float32)
    hs = 0.5 * s
    g = g * (1.5 - hs * g * g)
    g = g * (1.5 - hs * g * g)
    r = s * g
    rb = jnp.clip((r * (49.0 / _R_MAX)).astype(jnp.int32), 0, 49)

    ax = jnp.abs(x)
    ay = jnp.abs(y)
    hi = jnp.maximum(ax, ay)
    lo = jnp.minimum(ax, ay)
    rc = pl.reciprocal(hi, approx=True)
    rc = rc * (2.0 - hi * rc)
    t = lo * rc
    z = t * t
    a = _ATAN_C[5]
    for k in (4, 3, 2, 1, 0):
        a = a * z + _ATAN_C[k]
    a = a * t
    a = jnp.where(ay > ax, (math.pi / 2) - a, a)
    a = jnp.where(x < 0.0, math.pi - a, a)
    phi = jnp.where(y < 0.0, -a, a)
    pb = ((phi + math.pi) * (35.0 / (2.0 * math.pi))).astype(jnp.int32)
    pb = jnp.clip(pb, 0, 35)
    idx_ref[...] = rb * _NUM_PHI_BINS + pb


def _sine_body(_, pos_ref, wp_ref, out_ref):
    p = pos_ref[...]
    a2 = wp_ref[0:2, :]
    cc = wp_ref[2]
    wb = wp_ref[3]
    ph = wp_ref[4]
    th = lax.dot_general(p, a2, (((1,), (0,)), ((), ())),
                         preferred_element_type=jnp.float32)
    th = jnp.minimum(jnp.maximum(th + cc[None, :], 0.0), wb[None, :]) + ph[None, :]
    u = th - ((th + _RND) - _RND)
    z = u * u
    sv = _SIN_C[3]
    for k in (2, 1, 0):
        sv = sv * z + _SIN_C[k]
    out_ref[...] = sv * u


@functools.cache
def _make_sc_gather():
    mesh = plsc.VectorSubcoreMesh(core_axis_name="c", subcore_axis_name="s")
    return functools.partial(
        pl.kernel,
        out_type=jax.ShapeDtypeStruct((_N, 256), jnp.float32),
        mesh=mesh,
        scratch_types=[
            pltpu.VMEM_SHARED((_NUM_FUSED, 128), jnp.float32),
            pltpu.VMEM((_NCH, _CH), jnp.int32),
            pltpu.VMEM((_NBUF, _CH, 128), jnp.float32),
            pltpu.SemaphoreType.DMA,
            pltpu.SemaphoreType.DMA,
        ],
    )(_sc_gather_body)


def _sc_gather_body(table_hbm, idx_hbm, out_hbm, tab_sh, idx_v, rows_v, gsem, ssem):
    sid = lax.axis_index("s")
    wid = sid * _NC + lax.axis_index("c")
    row0 = wid * _BPW

    @pl.when(sid == 0)
    def _():
        pltpu.sync_copy(table_hbm, tab_sh)

    pltpu.sync_copy(idx_hbm.at[pl.ds(wid * _NCH, _NCH)], idx_v)
    plsc.subcore_barrier()

    def _gather(c, b):
        pltpu.async_copy(tab_sh.at[idx_v.at[c]], rows_v.at[b], gsem)

    def _gather_wait(c, b):
        pltpu.make_async_copy(tab_sh.at[idx_v.at[c]], rows_v.at[b], gsem).wait()

    def _store(c, b):
        dst = out_hbm.at[pl.ds(row0 + c * _CH, _CH), pl.ds(128, 128)]
        pltpu.async_copy(rows_v.at[b], dst, ssem)

    def _store_drain():
        dst = out_hbm.at[pl.ds(row0, _CH), pl.ds(128, 128)]
        pltpu.make_async_copy(rows_v.at[0], dst, ssem).wait()

    for j in range(_GAH):
        _gather(j, j)

    @pl.loop(0, _NCH, step=_NBUF)
    def _chunks(c0):
        for b in range(_NBUF):
            cc = c0 + b
            g = cc + _GAH

            @pl.when(cc >= _D0)
            def _():
                _store_drain()

            @pl.when(g < _NCH)
            def _():
                _gather(g, (b + _GAH) % _NBUF)

            _gather_wait(cc, b)
            _store(cc, b)

    for _ in range(_D0):
        _store_drain()


def _fused_table(r_embed, phi_embed):
    return jnp.concatenate(
        [
            jnp.broadcast_to(r_embed[:, None, :], (_NUM_R_BINS, _NUM_PHI_BINS, 64)),
            jnp.broadcast_to(phi_embed[None, :, :], (_NUM_R_BINS, _NUM_PHI_BINS, 64)),
        ],
        axis=-1,
    ).reshape(_NUM_FUSED, 128)


def kernel(positions, r_embed, phi_embed):
    pos2 = positions.reshape(_N, 2)
    xcol = positions[..., 0].reshape(_N // 128, 128)
    ycol = positions[..., 1].reshape(_N // 128, 128)

    idx2 = pl.pallas_call(
        _bins_body,
        grid=(_N // (_RA * 128),),
        in_specs=[
            pl.BlockSpec((_RA, 128), lambda i: (i, 0)),
            pl.BlockSpec((_RA, 128), lambda i: (i, 0)),
        ],
        out_specs=pl.BlockSpec((_RA, 128), lambda i: (i, 0)),
        out_shape=jax.ShapeDtypeStruct((_N // 128, 128), jnp.int32),
    )(xcol, ycol)

    comb = _make_sc_gather()(_fused_table(r_embed, phi_embed), idx2)

    comb = pl.pallas_call(
        _sine_body,
        grid=(_N // _BN_SINE,),
        in_specs=[
            pl.BlockSpec(memory_space=pl.ANY),
            pl.BlockSpec((_BN_SINE, 2), lambda i: (i, 0)),
            pl.BlockSpec((5, 128), lambda i: (0, 0)),
        ],
        out_specs=pl.BlockSpec((_BN_SINE, 128), lambda i: (i, 0)),
        out_shape=jax.ShapeDtypeStruct((_N, 256), jnp.float32),
        input_output_aliases={0: 0},
    )(comb, pos2, jnp.asarray(_WP_CONST))

    return comb.reshape(_BATCH, _SEQ, 256)


# speedup vs baseline: 3.3376x; 1.1404x over previous
import functools
import math

import numpy as np
import jax
import jax.numpy as jnp
from jax import lax
from jax.experimental import pallas as pl
from jax.experimental.pallas import tpu as pltpu
from jax.experimental.pallas import tpu_sc as plsc

_BATCH, _SEQ = 16, 8192
_N = _BATCH * _SEQ
_TEMPERATURE = 10000.0
_SCALE = 2.0 * math.pi
_R_MAX = 6000.0
_NUM_R_BINS = 50
_NUM_PHI_BINS = 36
_NUM_FUSED = _NUM_R_BINS * _NUM_PHI_BINS

_NC, _NS = 2, 16
_NW = _NC * _NS
_BPW = _N // _NW
_CH = 128
_NCH = _BPW // _CH
_NBUF = 4
_GAH = 2
_D0 = _NBUF - _GAH

_RA = 64
_BN_SINE = 4096

_ATAN_C = (0.9999798536300659, -0.3326554298400879, 0.1936698853969574,
           -0.11664997786283493, 0.05282219499349594, -0.011769973672926426)
_SIN_C = (6.278553009033203, -41.0910758972168, 77.90902709960938,
          -56.037471771240234)
_RND = 12582912.0


def _sine_consts():
    i = np.arange(64)
    dim_t = _TEMPERATURE ** (2.0 * np.floor(i / 2.0) / 64.0)
    w_half = 1.0 / dim_t
    ph_half = np.where(i % 2 == 1, 0.25, 0.0)
    w = np.concatenate([w_half, w_half])
    ph = np.concatenate([ph_half, ph_half])
    ax = np.where(np.arange(128) < 64, w / 6000.0, 0.0)
    ay = np.where(np.arange(128) < 64, 0.0, w / 4000.0)
    cc = w / 2.0
    return np.stack([ax, ay, cc, w, ph]).astype(np.float32)


_WP_CONST = _sine_consts()


def _bins_body(x_ref, y_ref, idx_ref):
    x = x_ref[...]
    y = y_ref[...]
    s = x * x + y * y
    i = lax.bitcast_convert_type(s, jnp.int32)
    i = 0x5F3759DF - lax.shift_right_logical(i, 1)
    g = lax.bitcast_convert_type(i, jnp.float32)
    hs = 0.5 * s
    g = g * (1.5 - hs * g * g)
    g = g * (1.5 - hs * g * g)
    r = s * g
    rb = jnp.clip((r * (49.0 / _R_MAX)).astype(jnp.int32), 0, 49)

    ax = jnp.abs(x)
    ay = jnp.abs(y)
    hi = jnp.maximum(ax, ay)
    lo = jnp.minimum(ax, ay)
    rc = pl.reciprocal(hi, approx=True)
    rc = rc * (2.0 - hi * rc)
    t = lo * rc
    z = t * t
    a = _ATAN_C[5]
    for k in (4, 3, 2, 1, 0):
        a = a * z + _ATAN_C[k]
    a = a * t
    a = jnp.where(ay > ax, (math.pi / 2) - a, a)
    a = jnp.where(x < 0.0, math.pi - a, a)
    phi = jnp.where(y < 0.0, -a, a)
    pb = ((phi + math.pi) * (35.0 / (2.0 * math.pi))).astype(jnp.int32)
    pb = jnp.clip(pb, 0, 35)
    idx_ref[...] = rb * _NUM_PHI_BINS + pb


def _sine_body(_, pos_ref, wp_ref, out_ref):
    p = pos_ref[...]
    a2 = wp_ref[0:2, :]
    cc = wp_ref[2]
    wb = wp_ref[3]
    ph = wp_ref[4]
    th = lax.dot_general(p, a2, (((1,), (0,)), ((), ())),
                         preferred_element_type=jnp.float32)
    th = jnp.minimum(jnp.maximum(th + cc[None, :], 0.0), wb[None, :]) + ph[None, :]
    u = th - ((th + _RND) - _RND)
    z = u * u
    sv = _SIN_C[3]
    for k in (2, 1, 0):
        sv = sv * z + _SIN_C[k]
    out_ref[...] = sv * u


@functools.cache
def _make_sc_gather():
    mesh = plsc.VectorSubcoreMesh(core_axis_name="c", subcore_axis_name="s")
    return functools.partial(
        pl.kernel,
        out_type=jax.ShapeDtypeStruct((_N, 256), jnp.float32),
        mesh=mesh,
        scratch_types=[
            pltpu.VMEM_SHARED((_NUM_FUSED, 128), jnp.float32),
            pltpu.VMEM((_NCH, _CH), jnp.int32),
            pltpu.VMEM((_NBUF, _CH, 128), jnp.float32),
            pltpu.SemaphoreType.DMA,
            pltpu.SemaphoreType.DMA,
        ],
    )(_sc_gather_body)


def _sc_gather_body(table_hbm, idx_hbm, out_hbm, tab_sh, idx_v, rows_v, gsem, ssem):
    sid = lax.axis_index("s")
    wid = sid * _NC + lax.axis_index("c")
    row0 = wid * _BPW

    @pl.when(sid == 0)
    def _():
        pltpu.sync_copy(table_hbm, tab_sh)

    pltpu.sync_copy(idx_hbm.at[pl.ds(wid * _NCH, _NCH)], idx_v)
    plsc.subcore_barrier()

    def _gather(c, b):
        pltpu.async_copy(tab_sh.at[idx_v.at[c]], rows_v.at[b], gsem)

    def _gather_wait(c, b):
        pltpu.make_async_copy(tab_sh.at[idx_v.at[c]], rows_v.at[b], gsem).wait()

    def _store(c, b):
        dst = out_hbm.at[pl.ds(row0 + c * _CH, _CH), pl.ds(128, 128)]
        pltpu.async_copy(rows_v.at[b], dst, ssem)

    def _store_drain():
        dst = out_hbm.at[pl.ds(row0, _CH), pl.ds(128, 128)]
        pltpu.make_async_copy(rows_v.at[0], dst, ssem).wait()

    for j in range(_GAH):
        _gather(j, j)

    @pl.loop(0, _NCH, step=_NBUF)
    def _chunks(c0):
        for b in range(_NBUF):
            cc = c0 + b
            g = cc + _GAH

            @pl.when(cc >= _D0)
            def _():
                _store_drain()

            @pl.when(g < _NCH)
            def _():
                _gather(g, (b + _GAH) % _NBUF)

            _gather_wait(cc, b)
            _store(cc, b)

    for _ in range(_D0):
        _store_drain()


def _fused_table(r_embed, phi_embed):
    return jnp.concatenate(
        [
            jnp.broadcast_to(r_embed[:, None, :], (_NUM_R_BINS, _NUM_PHI_BINS, 64)),
            jnp.broadcast_to(phi_embed[None, :, :], (_NUM_R_BINS, _NUM_PHI_BINS, 64)),
        ],
        axis=-1,
    ).reshape(_NUM_FUSED, 128)


def kernel(positions, r_embed, phi_embed):
    pos2 = positions.reshape(_N, 2)
    xcol = positions[..., 0].reshape(_N // 128, 128)
    ycol = positions[..., 1].reshape(_N // 128, 128)

    idx2 = pl.pallas_call(
        _bins_body,
        grid=(_N // (_RA * 128),),
        in_specs=[
            pl.BlockSpec((_RA, 128), lambda i: (i, 0)),
            pl.BlockSpec((_RA, 128), lambda i: (i, 0)),
        ],
        out_specs=pl.BlockSpec((_RA, 128), lambda i: (i, 0)),
        out_shape=jax.ShapeDtypeStruct((_N // 128, 128), jnp.int32),
    )(xcol, ycol)

    comb = _make_sc_gather()(_fused_table(r_embed, phi_embed), idx2)

    comb = pl.pallas_call(
        _sine_body,
        grid=(_N // _BN_SINE,),
        in_specs=[
            pl.BlockSpec(memory_space=pl.ANY),
            pl.BlockSpec((_BN_SINE, 2), lambda i: (i, 0)),
            pl.BlockSpec((5, 128), lambda i: (0, 0)),
        ],
        out_specs=pl.BlockSpec((_BN_SINE, 128), lambda i: (i, 0)),
        out_shape=jax.ShapeDtypeStruct((_N, 256), jnp.float32),
        input_output_aliases={0: 0},
    )(comb, pos2, jnp.asarray(_WP_CONST))

    return comb.reshape(_BATCH, _SEQ, 256)


# speedup vs baseline: 3.5689x; 1.0693x over previous
import functools
import math

import numpy as np
import jax
import jax.numpy as jnp
from jax import lax
from jax.experimental import pallas as pl
from jax.experimental.pallas import tpu as pltpu
from jax.experimental.pallas import tpu_sc as plsc

_BATCH, _SEQ = 16, 8192
_N = _BATCH * _SEQ
_TEMPERATURE = 10000.0
_SCALE = 2.0 * math.pi
_R_MAX = 6000.0
_NUM_R_BINS = 50
_NUM_PHI_BINS = 36
_NUM_FUSED = _NUM_R_BINS * _NUM_PHI_BINS

_NC, _NS = 2, 16
_NW = _NC * _NS
_BPW = _N // _NW
_CH = 128
_NCH = _BPW // _CH
_NBUF = 4
_GAH = 2
_D0 = _NBUF - _GAH

_RA = 64
_BN_SINE = 8192

_ATAN_C = (0.9999798536300659, -0.3326554298400879, 0.1936698853969574,
           -0.11664997786283493, 0.05282219499349594, -0.011769973672926426)
_SIN_C = (6.278553009033203, -41.0910758972168, 77.90902709960938,
          -56.037471771240234)
_RND = 12582912.0


def _sine_consts():
    i = np.arange(64)
    dim_t = _TEMPERATURE ** (2.0 * np.floor(i / 2.0) / 64.0)
    w_half = 1.0 / dim_t
    ph_half = np.where(i % 2 == 1, 0.25, 0.0)
    w = np.concatenate([w_half, w_half])
    ph = np.concatenate([ph_half, ph_half])
    ax = np.where(np.arange(128) < 64, w / 6000.0, 0.0)
    ay = np.where(np.arange(128) < 64, 0.0, w / 4000.0)
    cc = w / 2.0
    return np.stack([ax, ay, cc, w, ph]).astype(np.float32)


_WP_CONST = _sine_consts()


def _bins_body(x_ref, y_ref, idx_ref):
    x = x_ref[...]
    y = y_ref[...]
    s = x * x + y * y
    i = lax.bitcast_convert_type(s, jnp.int32)
    i = 0x5F3759DF - lax.shift_right_logical(i, 1)
    g = lax.bitcast_convert_type(i, jnp.float32)
    hs = 0.5 * s
    g = g * (1.5 - hs * g * g)
    g = g * (1.5 - hs * g * g)
    r = s * g
    rb = jnp.clip((r * (49.0 / _R_MAX)).astype(jnp.int32), 0, 49)

    ax = jnp.abs(x)
    ay = jnp.abs(y)
    hi = jnp.maximum(ax, ay)
    lo = jnp.minimum(ax, ay)
    rc = pl.reciprocal(hi, approx=True)
    rc = rc * (2.0 - hi * rc)
    t = lo * rc
    z = t * t
    a = _ATAN_C[5]
    for k in (4, 3, 2, 1, 0):
        a = a * z + _ATAN_C[k]
    a = a * t
    a = jnp.where(ay > ax, (math.pi / 2) - a, a)
    a = jnp.where(x < 0.0, math.pi - a, a)
    phi = jnp.where(y < 0.0, -a, a)
    pb = ((phi + math.pi) * (35.0 / (2.0 * math.pi))).astype(jnp.int32)
    pb = jnp.clip(pb, 0, 35)
    idx_ref[...] = rb * _NUM_PHI_BINS + pb


def _sine_body(_, pos_ref, wp_ref, out_ref):
    p = pos_ref[...]
    a2 = wp_ref[0:2, :]
    cc = wp_ref[2]
    wb = wp_ref[3]
    ph = wp_ref[4]
    th = lax.dot_general(p, a2, (((1,), (0,)), ((), ())),
                         preferred_element_type=jnp.float32)
    th = jnp.minimum(jnp.maximum(th + cc[None, :], 0.0), wb[None, :]) + ph[None, :]
    u = th - ((th + _RND) - _RND)
    z = u * u
    sv = _SIN_C[3]
    for k in (2, 1, 0):
        sv = sv * z + _SIN_C[k]
    out_ref[...] = sv * u


@functools.cache
def _make_sc_gather():
    mesh = plsc.VectorSubcoreMesh(core_axis_name="c", subcore_axis_name="s")
    return functools.partial(
        pl.kernel,
        out_type=jax.ShapeDtypeStruct((_N, 256), jnp.float32),
        mesh=mesh,
        scratch_types=[
            pltpu.VMEM_SHARED((_NUM_FUSED, 128), jnp.float32),
            pltpu.VMEM((_NCH, _CH), jnp.int32),
            pltpu.VMEM((_NBUF, _CH, 128), jnp.float32),
            pltpu.SemaphoreType.DMA,
            pltpu.SemaphoreType.DMA,
        ],
    )(_sc_gather_body)


def _sc_gather_body(table_hbm, idx_hbm, out_hbm, tab_sh, idx_v, rows_v, gsem, ssem):
    sid = lax.axis_index("s")
    wid = sid * _NC + lax.axis_index("c")
    row0 = wid * _BPW

    @pl.when(sid == 0)
    def _():
        pltpu.sync_copy(table_hbm, tab_sh)

    pltpu.sync_copy(idx_hbm.at[pl.ds(wid * _NCH, _NCH)], idx_v)
    plsc.subcore_barrier()

    def _gather(c, b):
        pltpu.async_copy(tab_sh.at[idx_v.at[c]], rows_v.at[b], gsem)

    def _gather_wait(c, b):
        pltpu.make_async_copy(tab_sh.at[idx_v.at[c]], rows_v.at[b], gsem).wait()

    def _store(c, b):
        dst = out_hbm.at[pl.ds(row0 + c * _CH, _CH), pl.ds(128, 128)]
        pltpu.async_copy(rows_v.at[b], dst, ssem)

    def _store_drain():
        dst = out_hbm.at[pl.ds(row0, _CH), pl.ds(128, 128)]
        pltpu.make_async_copy(rows_v.at[0], dst, ssem).wait()

    for j in range(_GAH):
        _gather(j, j)

    @pl.loop(0, _NCH, step=_NBUF)
    def _chunks(c0):
        for b in range(_NBUF):
            cc = c0 + b
            g = cc + _GAH

            @pl.when(cc >= _D0)
            def _():
                _store_drain()

            @pl.when(g < _NCH)
            def _():
                _gather(g, (b + _GAH) % _NBUF)

            _gather_wait(cc, b)
            _store(cc, b)

    for _ in range(_D0):
        _store_drain()


def _fused_table(r_embed, phi_embed):
    return jnp.concatenate(
        [
            jnp.broadcast_to(r_embed[:, None, :], (_NUM_R_BINS, _NUM_PHI_BINS, 64)),
            jnp.broadcast_to(phi_embed[None, :, :], (_NUM_R_BINS, _NUM_PHI_BINS, 64)),
        ],
        axis=-1,
    ).reshape(_NUM_FUSED, 128)


def kernel(positions, r_embed, phi_embed):
    pos2 = positions.reshape(_N, 2)
    xcol = positions[..., 0].reshape(_N // 128, 128)
    ycol = positions[..., 1].reshape(_N // 128, 128)

    idx2 = pl.pallas_call(
        _bins_body,
        grid=(_N // (_RA * 128),),
        in_specs=[
            pl.BlockSpec((_RA, 128), lambda i: (i, 0)),
            pl.BlockSpec((_RA, 128), lambda i: (i, 0)),
        ],
        out_specs=pl.BlockSpec((_RA, 128), lambda i: (i, 0)),
        out_shape=jax.ShapeDtypeStruct((_N // 128, 128), jnp.int32),
    )(xcol, ycol)

    comb = _make_sc_gather()(_fused_table(r_embed, phi_embed), idx2)

    comb = pl.pallas_call(
        _sine_body,
        grid=(_N // _BN_SINE,),
        in_specs=[
            pl.BlockSpec(memory_space=pl.ANY),
            pl.BlockSpec((_BN_SINE, 2), lambda i: (i, 0)),
            pl.BlockSpec((5, 128), lambda i: (0, 0)),
        ],
        out_specs=pl.BlockSpec((_BN_SINE, 128), lambda i: (i, 0)),
        out_shape=jax.ShapeDtypeStruct((_N, 256), jnp.float32),
        input_output_aliases={0: 0},
    )(comb, pos2, jnp.asarray(_WP_CONST))

    return comb.reshape(_BATCH, _SEQ, 256)


# speedup vs baseline: 3.6596x; 1.0254x over previous
import functools
import math

import numpy as np
import jax
import jax.numpy as jnp
from jax import lax
from jax.experimental import pallas as pl
from jax.experimental.pallas import tpu as pltpu
from jax.experimental.pallas import tpu_sc as plsc

_BATCH, _SEQ = 16, 8192
_N = _BATCH * _SEQ
_TEMPERATURE = 10000.0
_SCALE = 2.0 * math.pi
_R_MAX = 6000.0
_NUM_R_BINS = 50
_NUM_PHI_BINS = 36
_NUM_FUSED = _NUM_R_BINS * _NUM_PHI_BINS

_NC, _NS = 2, 16
_NW = _NC * _NS
_BPW = _N // _NW
_CH = 128
_NCH = _BPW // _CH
_NBUF = 4
_GAH = 2
_D0 = _NBUF - _GAH

_RA = 64
_BN_SINE = 16384

_ATAN_C = (0.9999798536300659, -0.3326554298400879, 0.1936698853969574,
           -0.11664997786283493, 0.05282219499349594, -0.011769973672926426)
_SIN_C = (6.278553009033203, -41.0910758972168, 77.90902709960938,
          -56.037471771240234)
_RND = 12582912.0


def _sine_consts():
    i = np.arange(64)
    dim_t = _TEMPERATURE ** (2.0 * np.floor(i / 2.0) / 64.0)
    w_half = 1.0 / dim_t
    ph_half = np.where(i % 2 == 1, 0.25, 0.0)
    w = np.concatenate([w_half, w_half])
    ph = np.concatenate([ph_half, ph_half])
    ax = np.where(np.arange(128) < 64, w / 6000.0, 0.0)
    ay = np.where(np.arange(128) < 64, 0.0, w / 4000.0)
    cc = w / 2.0
    return np.stack([ax, ay, cc, w, ph]).astype(np.float32)


_WP_CONST = _sine_consts()


def _bins_body(x_ref, y_ref, idx_ref):
    x = x_ref[...]
    y = y_ref[...]
    s = x * x + y * y
    i = lax.bitcast_convert_type(s, jnp.int32)
    i = 0x5F3759DF - lax.shift_right_logical(i, 1)
    g = lax.bitcast_convert_type(i, jnp.float32)
    hs = 0.5 * s
    g = g * (1.5 - hs * g * g)
    g = g * (1.5 - hs * g * g)
    r = s * g
    rb = jnp.clip((r * (49.0 / _R_MAX)).astype(jnp.int32), 0, 49)

    ax = jnp.abs(x)
    ay = jnp.abs(y)
    hi = jnp.maximum(ax, ay)
    lo = jnp.minimum(ax, ay)
    rc = pl.reciprocal(hi, approx=True)
    rc = rc * (2.0 - hi * rc)
    t = lo * rc
    z = t * t
    a = _ATAN_C[5]
    for k in (4, 3, 2, 1, 0):
        a = a * z + _ATAN_C[k]
    a = a * t
    a = jnp.where(ay > ax, (math.pi / 2) - a, a)
    a = jnp.where(x < 0.0, math.pi - a, a)
    phi = jnp.where(y < 0.0, -a, a)
    pb = ((phi + math.pi) * (35.0 / (2.0 * math.pi))).astype(jnp.int32)
    pb = jnp.clip(pb, 0, 35)
    idx_ref[...] = rb * _NUM_PHI_BINS + pb


def _sine_body(_, pos_ref, wp_ref, out_ref):
    p = pos_ref[...]
    a2 = wp_ref[0:2, :]
    cc = wp_ref[2]
    wb = wp_ref[3]
    ph = wp_ref[4]
    th = lax.dot_general(p, a2, (((1,), (0,)), ((), ())),
                         preferred_element_type=jnp.float32)
    th = jnp.minimum(jnp.maximum(th + cc[None, :], 0.0), wb[None, :]) + ph[None, :]
    u = th - ((th + _RND) - _RND)
    z = u * u
    sv = _SIN_C[3]
    for k in (2, 1, 0):
        sv = sv * z + _SIN_C[k]
    out_ref[...] = sv * u


@functools.cache
def _make_sc_gather():
    mesh = plsc.VectorSubcoreMesh(core_axis_name="c", subcore_axis_name="s")
    return functools.partial(
        pl.kernel,
        out_type=jax.ShapeDtypeStruct((_N, 256), jnp.float32),
        mesh=mesh,
        scratch_types=[
            pltpu.VMEM_SHARED((_NUM_FUSED, 128), jnp.float32),
            pltpu.VMEM((_NCH, _CH), jnp.int32),
            pltpu.VMEM((_NBUF, _CH, 128), jnp.float32),
            pltpu.SemaphoreType.DMA,
            pltpu.SemaphoreType.DMA,
        ],
    )(_sc_gather_body)


def _sc_gather_body(table_hbm, idx_hbm, out_hbm, tab_sh, idx_v, rows_v, gsem, ssem):
    sid = lax.axis_index("s")
    wid = sid * _NC + lax.axis_index("c")
    row0 = wid * _BPW

    @pl.when(sid == 0)
    def _():
        pltpu.sync_copy(table_hbm, tab_sh)

    pltpu.sync_copy(idx_hbm.at[pl.ds(wid * _NCH, _NCH)], idx_v)
    plsc.subcore_barrier()

    def _gather(c, b):
        pltpu.async_copy(tab_sh.at[idx_v.at[c]], rows_v.at[b], gsem)

    def _gather_wait(c, b):
        pltpu.make_async_copy(tab_sh.at[idx_v.at[c]], rows_v.at[b], gsem).wait()

    def _store(c, b):
        dst = out_hbm.at[pl.ds(row0 + c * _CH, _CH), pl.ds(128, 128)]
        pltpu.async_copy(rows_v.at[b], dst, ssem)

    def _store_drain():
        dst = out_hbm.at[pl.ds(row0, _CH), pl.ds(128, 128)]
        pltpu.make_async_copy(rows_v.at[0], dst, ssem).wait()

    for j in range(_GAH):
        _gather(j, j)

    @pl.loop(0, _NCH, step=_NBUF)
    def _chunks(c0):
        for b in range(_NBUF):
            cc = c0 + b
            g = cc + _GAH

            @pl.when(cc >= _D0)
            def _():
                _store_drain()

            @pl.when(g < _NCH)
            def _():
                _gather(g, (b + _GAH) % _NBUF)

            _gather_wait(cc, b)
            _store(cc, b)

    for _ in range(_D0):
        _store_drain()


def _fused_table(r_embed, phi_embed):
    return jnp.concatenate(
        [
            jnp.broadcast_to(r_embed[:, None, :], (_NUM_R_BINS, _NUM_PHI_BINS, 64)),
            jnp.broadcast_to(phi_embed[None, :, :], (_NUM_R_BINS, _NUM_PHI_BINS, 64)),
        ],
        axis=-1,
    ).reshape(_NUM_FUSED, 128)


def kernel(positions, r_embed, phi_embed):
    pos2 = positions.reshape(_N, 2)
    xcol = positions[..., 0].reshape(_N // 128, 128)
    ycol = positions[..., 1].reshape(_N // 128, 128)

    idx2 = pl.pallas_call(
        _bins_body,
        grid=(_N // (_RA * 128),),
        in_specs=[
            pl.BlockSpec((_RA, 128), lambda i: (i, 0)),
            pl.BlockSpec((_RA, 128), lambda i: (i, 0)),
        ],
        out_specs=pl.BlockSpec((_RA, 128), lambda i: (i, 0)),
        out_shape=jax.ShapeDtypeStruct((_N // 128, 128), jnp.int32),
    )(xcol, ycol)

    comb = _make_sc_gather()(_fused_table(r_embed, phi_embed), idx2)

    comb = pl.pallas_call(
        _sine_body,
        grid=(_N // _BN_SINE,),
        in_specs=[
            pl.BlockSpec(memory_space=pl.ANY),
            pl.BlockSpec((_BN_SINE, 2), lambda i: (i, 0)),
            pl.BlockSpec((5, 128), lambda i: (0, 0)),
        ],
        out_specs=pl.BlockSpec((_BN_SINE, 128), lambda i: (i, 0)),
        out_shape=jax.ShapeDtypeStruct((_N, 256), jnp.float32),
        input_output_aliases={0: 0},
    )(comb, pos2, jnp.asarray(_WP_CONST))

    return comb.reshape(_BATCH, _SEQ, 256)
